# baseline (speedup 1.0000x reference)
import functools

import jax
import jax.numpy as jnp
from jax import lax
from jax.experimental import pallas as pl
from jax.experimental.pallas import tpu as pltpu
from jax.experimental.pallas import tpu_sc as plsc

N_USER = 5000
N_ITEM = 5000
N = N_USER + N_ITEM
D_IN = 256
D_GCN = 256
D_DENSE = 128
NNZ = 160000

NUM_CORES = 2
NUM_SUBCORES = 16
DH = D_GCN // 2

EDGES_PER_TILE = NNZ // NUM_SUBCORES
CHUNK = 80
NUM_CHUNKS = EDGES_PER_TILE // CHUNK
ZCHUNK = 80
NZROWCH = N // ZCHUNK


_RB = 1000


def _mm_kernel(x_ref, w0_ref, w1_ref, h0_ref, h1_ref):
    x = x_ref[...].astype(jnp.bfloat16)
    w0 = w0_ref[...]
    w1 = w1_ref[...]
    f32 = jnp.float32
    h0_ref[0] = jax.lax.dot(x, w0.astype(jnp.bfloat16),
                            preferred_element_type=f32)
    h1_ref[0] = jax.lax.dot(x, (w0 + w1).astype(jnp.bfloat16),
                            preferred_element_type=f32)


def _features(x, w0, w1):
    grid = (N // _RB, NUM_CORES)
    return pl.pallas_call(
        _mm_kernel,
        grid=grid,
        in_specs=[
            pl.BlockSpec((_RB, D_IN), lambda r, c: (r, 0)),
            pl.BlockSpec((D_IN, DH), lambda r, c: (0, c)),
            pl.BlockSpec((D_IN, DH), lambda r, c: (0, c)),
        ],
        out_specs=[
            pl.BlockSpec((1, _RB, DH), lambda r, c: (c, r, 0)),
            pl.BlockSpec((1, _RB, DH), lambda r, c: (c, r, 0)),
        ],
        out_shape=[
            jax.ShapeDtypeStruct((NUM_CORES, N, DH), jnp.float32),
            jax.ShapeDtypeStruct((NUM_CORES, N, DH), jnp.float32),
        ],
    )(x, w0, w1)


NBUF = 4
NPBUF = 6
_UNROLL = 12


def _spmm_body(h0, h1, p0, p1, out, pbufs, bufs, psems, gsems, ssems, emb_sh):
    cid = lax.axis_index("c")
    sid = lax.axis_index("s")
    pbase = sid * NUM_CHUNKS

    bufs[0][...] = jnp.zeros_like(bufs[0])
    for t in range(-(-NZROWCH // NUM_SUBCORES)):
        j = sid + t * NUM_SUBCORES

        @pl.when(j < NZROWCH)
        def _():
            pltpu.sync_copy(bufs[0], emb_sh.at[pl.ds(j * ZCHUNK, ZCHUNK)])

    plsc.subcore_barrier()

    def do_support(h_ref, packed):
        def issue_pload(c, p):
            pltpu.async_copy(packed.at[pbase + c], pbufs[p], psems[p])

        def wait_pload(c, p):
            pltpu.make_async_copy(packed.at[pbase + c], pbufs[p],
                                  psems[p]).wait()

        def issue_gather(c, b, p):
            pltpu.async_copy(h_ref.at[pbufs[p].at[0]], bufs[b], gsems[b])

        def wait_gather(c, b, p):
            pltpu.make_async_copy(h_ref.at[pbufs[p].at[0]], bufs[b],
                                  gsems[b]).wait()

        def issue_scatter(c, b, p):
            pltpu.async_copy(bufs[b], emb_sh.at[pbufs[p].at[1]], ssems[b],
                             add=True)

        def wait_scatter(c, b, p):
            pltpu.make_async_copy(bufs[b], emb_sh.at[pbufs[p].at[1]],
                                  ssems[b]).wait()

        def scale(c, b, p):
            buf = bufs[b]
            vrow = pbufs[p]

            def scale_row(e, _):
                vi = vrow[2, pl.ds(e, 16)][0]
                val = jax.lax.bitcast_convert_type(vi, jnp.float32)
                for k in range(DH // 16):
                    sl = pl.ds(k * 16, 16)
                    buf[e, sl] = buf[e, sl] * val
                return ()

            lax.fori_loop(0, CHUNK, scale_row, ())

        def step(c, u, guarded, wait_sc, do_pload, do_gather):
            if wait_sc:
                if guarded:
                    @pl.when(c >= 2)
                    def _():
                        wait_scatter(c - 2, (u + 2) % NBUF, (u + 4) % NPBUF)
                else:
                    wait_scatter(c - 2, (u + 2) % NBUF, (u + 4) % NPBUF)
            if do_pload:
                issue_pload(c + 4, (u + 4) % NPBUF)
            if do_gather:
                wait_pload(c + 2, (u + 2) % NPBUF)
                issue_gather(c + 2, (u + 2) % NBUF, (u + 2) % NPBUF)
            wait_gather(c, u % NBUF, u % NPBUF)
            scale(c, u % NBUF, u % NPBUF)
            issue_scatter(c, u % NBUF, u % NPBUF)

        for j in range(4):
            issue_pload(j, j)
        for j in range(2):
            wait_pload(j, j)
            issue_gather(j, j, j)

        def group(i12, _):
            for u in range(_UNROLL):
                step(i12 * _UNROLL + u, u, True, True, True, True)
            return ()

        n_main = (NUM_CHUNKS - 4) // _UNROLL
        lax.fori_loop(0, n_main, group, ())
        for c in range(n_main * _UNROLL, NUM_CHUNKS):
            step(c, c % _UNROLL, False, True,
                 c + 4 < NUM_CHUNKS, c + 2 < NUM_CHUNKS)
        for c in range(NUM_CHUNKS - 2, NUM_CHUNKS):
            wait_scatter(c, c % NBUF, c % NPBUF)

    do_support(h0.at[cid], p0)
    do_support(h1.at[cid], p1)

    plsc.subcore_barrier()
    for t in range(-(-NZROWCH // NUM_SUBCORES)):
        j = sid + t * NUM_SUBCORES

        @pl.when(j < NZROWCH)
        def _():
            sl = pl.ds(j * ZCHUNK, ZCHUNK)
            pltpu.sync_copy(emb_sh.at[sl], out.at[cid].at[sl])


def _pack(rows, cols, vals):
    nch = NNZ // CHUNK
    return jnp.stack(
        [cols.reshape(nch, CHUNK),
         rows.reshape(nch, CHUNK),
         jax.lax.bitcast_convert_type(vals, jnp.int32).reshape(nch, CHUNK),
         jnp.zeros((nch, CHUNK), jnp.int32)],
        axis=1)


def _spmm(h0, h1, r0, c0, v0, r1, c1, v1):
    mesh = plsc.VectorSubcoreMesh(core_axis_name="c", subcore_axis_name="s")
    p0 = _pack(r0, c0, v0)
    p1 = _pack(r1, c1, v1)
    return pl.kernel(
        _spmm_body,
        out_type=jax.ShapeDtypeStruct((NUM_CORES, N, DH), jnp.float32),
        mesh=mesh,
        scratch_types=[
            [pltpu.VMEM((4, CHUNK), jnp.int32)] * NPBUF,
            [pltpu.VMEM((CHUNK, DH), jnp.float32)] * NBUF,
            [pltpu.SemaphoreType.DMA] * NPBUF,
            [pltpu.SemaphoreType.DMA] * NBUF,
            [pltpu.SemaphoreType.DMA] * NBUF,
            pltpu.VMEM_SHARED((N, DH), jnp.float32),
        ],
    )(h0, h1, p0, p1)


def _out_kernel(emb_ref, wd_ref, o_ref):
    e0 = jnp.maximum(emb_ref[0], 0.0)
    e1 = jnp.maximum(emb_ref[1], 0.0)
    wd0 = wd_ref[:, :DH]
    wd1 = wd_ref[:, DH:]
    dn = (((1,), (1,)), ((), ()))
    o_ref[...] = (jax.lax.dot_general(e0, wd0, dn)
                  + jax.lax.dot_general(e1, wd1, dn))


def _project(emb, wd):
    return pl.pallas_call(
        _out_kernel,
        grid=(N // _RB,),
        in_specs=[
            pl.BlockSpec((NUM_CORES, _RB, DH), lambda r: (0, r, 0)),
            pl.BlockSpec((D_DENSE, D_GCN), lambda r: (0, 0)),
        ],
        out_specs=pl.BlockSpec((_RB, D_DENSE), lambda r: (r, 0)),
        out_shape=jax.ShapeDtypeStruct((N, D_DENSE), jnp.float32),
    )(emb, wd)


@jax.jit
def kernel(user_X, item_X, W0, W1, Wd, A0_rows, A0_cols, A0_vals,
           A1_rows, A1_cols, A1_vals):
    x = jnp.concatenate([user_X, item_X], axis=0)
    h0, h1 = _features(x, W0, W1)
    emb = _spmm(h0, h1, A0_rows, A0_cols, A0_vals, A1_rows, A1_cols, A1_vals)
    out = _project(emb, Wd)
    return out[:N_USER], out[N_USER:]


# speedup vs baseline: 1.0363x; 1.0363x over previous
import functools

import jax
import jax.numpy as jnp
from jax import lax
from jax.experimental import pallas as pl
from jax.experimental.pallas import tpu as pltpu
from jax.experimental.pallas import tpu_sc as plsc

N_USER = 5000
N_ITEM = 5000
N = N_USER + N_ITEM
D_IN = 256
D_GCN = 256
D_DENSE = 128
NNZ = 160000

NUM_CORES = 2
NUM_SUBCORES = 16
DH = D_GCN // 2

EDGES_PER_TILE = NNZ // NUM_SUBCORES
CHUNK = 80
NUM_CHUNKS = EDGES_PER_TILE // CHUNK
ZCHUNK = 80
NZROWCH = N // ZCHUNK


_RB = 1000


def _mm_kernel(x_ref, w0_ref, w1_ref, h0_ref, h1_ref):
    x = x_ref[...]
    w0 = w0_ref[...]
    w01 = w0 + w1_ref[...]
    h0 = jax.lax.dot(x, w0)
    h1 = jax.lax.dot(x, w01)
    h0_ref[0] = h0[:, :DH]
    h0_ref[1] = h0[:, DH:]
    h1_ref[0] = h1[:, :DH]
    h1_ref[1] = h1[:, DH:]


def _features(x, w0, w1):
    return pl.pallas_call(
        _mm_kernel,
        grid=(N // _RB,),
        in_specs=[
            pl.BlockSpec((_RB, D_IN), lambda r: (r, 0)),
            pl.BlockSpec((D_IN, D_GCN), lambda r: (0, 0)),
            pl.BlockSpec((D_IN, D_GCN), lambda r: (0, 0)),
        ],
        out_specs=[
            pl.BlockSpec((NUM_CORES, _RB, DH), lambda r: (0, r, 0)),
            pl.BlockSpec((NUM_CORES, _RB, DH), lambda r: (0, r, 0)),
        ],
        out_shape=[
            jax.ShapeDtypeStruct((NUM_CORES, N, DH), jnp.float32),
            jax.ShapeDtypeStruct((NUM_CORES, N, DH), jnp.float32),
        ],
    )(x, w0, w1)


NBUF = 4
NPBUF = 6
_UNROLL = 12


def _spmm_body(h0, h1, p0, p1, out, pbufs, bufs, psems, gsems, ssems, emb_sh):
    cid = lax.axis_index("c")
    sid = lax.axis_index("s")
    pbase = sid * NUM_CHUNKS

    bufs[0][...] = jnp.zeros_like(bufs[0])
    for t in range(-(-NZROWCH // NUM_SUBCORES)):
        j = sid + t * NUM_SUBCORES

        @pl.when(j < NZROWCH)
        def _():
            pltpu.sync_copy(bufs[0], emb_sh.at[pl.ds(j * ZCHUNK, ZCHUNK)])

    plsc.subcore_barrier()

    def do_support(h_ref, packed):
        def issue_pload(c, p):
            pltpu.async_copy(packed.at[pbase + c], pbufs[p], psems[p])

        def wait_pload(c, p):
            pltpu.make_async_copy(packed.at[pbase + c], pbufs[p],
                                  psems[p]).wait()

        def issue_gather(c, b, p):
            pltpu.async_copy(h_ref.at[pbufs[p].at[0]], bufs[b], gsems[b])

        def wait_gather(c, b, p):
            pltpu.make_async_copy(h_ref.at[pbufs[p].at[0]], bufs[b],
                                  gsems[b]).wait()

        def issue_scatter(c, b, p):
            pltpu.async_copy(bufs[b], emb_sh.at[pbufs[p].at[1]], ssems[b],
                             add=True)

        def wait_scatter(c, b, p):
            pltpu.make_async_copy(bufs[b], emb_sh.at[pbufs[p].at[1]],
                                  ssems[b]).wait()

        def scale(c, b, p):
            buf = bufs[b]
            vrow = pbufs[p]

            def scale_row(e, _):
                vi = vrow[2, pl.ds(e, 16)][0]
                val = jax.lax.bitcast_convert_type(vi, jnp.float32)
                for k in range(DH // 16):
                    sl = pl.ds(k * 16, 16)
                    buf[e, sl] = buf[e, sl] * val
                return ()

            lax.fori_loop(0, CHUNK, scale_row, ())

        def step(c, u, guarded, wait_sc, do_pload, do_gather):
            if wait_sc:
                if guarded:
                    @pl.when(c >= 2)
                    def _():
                        wait_scatter(c - 2, (u + 2) % NBUF, (u + 4) % NPBUF)
                else:
                    wait_scatter(c - 2, (u + 2) % NBUF, (u + 4) % NPBUF)
            if do_pload:
                issue_pload(c + 4, (u + 4) % NPBUF)
            if do_gather:
                wait_pload(c + 2, (u + 2) % NPBUF)
                issue_gather(c + 2, (u + 2) % NBUF, (u + 2) % NPBUF)
            wait_gather(c, u % NBUF, u % NPBUF)
            scale(c, u % NBUF, u % NPBUF)
            issue_scatter(c, u % NBUF, u % NPBUF)

        for j in range(4):
            issue_pload(j, j)
        for j in range(2):
            wait_pload(j, j)
            issue_gather(j, j, j)

        def group(i12, _):
            for u in range(_UNROLL):
                step(i12 * _UNROLL + u, u, True, True, True, True)
            return ()

        n_main = (NUM_CHUNKS - 4) // _UNROLL
        lax.fori_loop(0, n_main, group, ())
        for c in range(n_main * _UNROLL, NUM_CHUNKS):
            step(c, c % _UNROLL, False, True,
                 c + 4 < NUM_CHUNKS, c + 2 < NUM_CHUNKS)
        for c in range(NUM_CHUNKS - 2, NUM_CHUNKS):
            wait_scatter(c, c % NBUF, c % NPBUF)

    do_support(h0.at[cid], p0)
    do_support(h1.at[cid], p1)

    plsc.subcore_barrier()
    for t in range(-(-NZROWCH // NUM_SUBCORES)):
        j = sid + t * NUM_SUBCORES

        @pl.when(j < NZROWCH)
        def _():
            sl = pl.ds(j * ZCHUNK, ZCHUNK)
            pltpu.sync_copy(emb_sh.at[sl], out.at[cid].at[sl])


def _pack(rows, cols, vals):
    nch = NNZ // CHUNK
    return jnp.stack(
        [cols.reshape(nch, CHUNK),
         rows.reshape(nch, CHUNK),
         jax.lax.bitcast_convert_type(vals, jnp.int32).reshape(nch, CHUNK),
         jnp.zeros((nch, CHUNK), jnp.int32)],
        axis=1)


def _spmm(h0, h1, r0, c0, v0, r1, c1, v1):
    mesh = plsc.VectorSubcoreMesh(core_axis_name="c", subcore_axis_name="s")
    p0 = _pack(r0, c0, v0)
    p1 = _pack(r1, c1, v1)
    return pl.kernel(
        _spmm_body,
        out_type=jax.ShapeDtypeStruct((NUM_CORES, N, DH), jnp.float32),
        mesh=mesh,
        scratch_types=[
            [pltpu.VMEM((4, CHUNK), jnp.int32)] * NPBUF,
            [pltpu.VMEM((CHUNK, DH), jnp.float32)] * NBUF,
            [pltpu.SemaphoreType.DMA] * NPBUF,
            [pltpu.SemaphoreType.DMA] * NBUF,
            [pltpu.SemaphoreType.DMA] * NBUF,
            pltpu.VMEM_SHARED((N, DH), jnp.float32),
        ],
    )(h0, h1, p0, p1)


def _out_kernel(emb_ref, wd_ref, o_ref):
    e0 = jnp.maximum(emb_ref[0], 0.0)
    e1 = jnp.maximum(emb_ref[1], 0.0)
    wd0 = wd_ref[:, :DH]
    wd1 = wd_ref[:, DH:]
    dn = (((1,), (1,)), ((), ()))
    o_ref[...] = (jax.lax.dot_general(e0, wd0, dn)
                  + jax.lax.dot_general(e1, wd1, dn))


def _project(emb, wd):
    return pl.pallas_call(
        _out_kernel,
        grid=(N // _RB,),
        in_specs=[
            pl.BlockSpec((NUM_CORES, _RB, DH), lambda r: (0, r, 0)),
            pl.BlockSpec((D_DENSE, D_GCN), lambda r: (0, 0)),
        ],
        out_specs=pl.BlockSpec((_RB, D_DENSE), lambda r: (r, 0)),
        out_shape=jax.ShapeDtypeStruct((N, D_DENSE), jnp.float32),
    )(emb, wd)


@jax.jit
def kernel(user_X, item_X, W0, W1, Wd, A0_rows, A0_cols, A0_vals,
           A1_rows, A1_cols, A1_vals):
    x = jnp.concatenate([user_X, item_X], axis=0)
    h0, h1 = _features(x, W0, W1)
    emb = _spmm(h0, h1, A0_rows, A0_cols, A0_vals, A1_rows, A1_cols, A1_vals)
    out = _project(emb, Wd)
    return out[:N_USER], out[N_USER:]


# speedup vs baseline: 1.0366x; 1.0004x over previous
import jax
import jax.numpy as jnp
from jax import lax
from jax.experimental import pallas as pl
from jax.experimental.pallas import tpu as pltpu
from jax.experimental.pallas import tpu_sc as plsc

N_USER = 5000
N_ITEM = 5000
N = N_USER + N_ITEM
D_IN = 256
D_GCN = 256
D_DENSE = 128
NNZ = 160000

NUM_CORES = 2
NUM_SUBCORES = 16
DH = D_GCN // 2

EDGES_PER_TILE = NNZ // NUM_SUBCORES
CHUNK = 80
NUM_CHUNKS = EDGES_PER_TILE // CHUNK
ZCHUNK = 80
NZROWCH = N // ZCHUNK


_RB = 1000


def _mm_kernel(x_ref, w0_ref, w1_ref, h0_ref, h1_ref):
    x = x_ref[...]
    w0 = w0_ref[...]
    w01 = w0 + w1_ref[...]
    h0 = jax.lax.dot(x, w0)
    h1 = jax.lax.dot(x, w01)
    h0_ref[0] = h0[:, :DH]
    h0_ref[1] = h0[:, DH:]
    h1_ref[0] = h1[:, :DH]
    h1_ref[1] = h1[:, DH:]


def _features(x, w0, w1):
    return pl.pallas_call(
        _mm_kernel,
        grid=(N // _RB,),
        in_specs=[
            pl.BlockSpec((_RB, D_IN), lambda r: (r, 0)),
            pl.BlockSpec((D_IN, D_GCN), lambda r: (0, 0)),
            pl.BlockSpec((D_IN, D_GCN), lambda r: (0, 0)),
        ],
        out_specs=[
            pl.BlockSpec((NUM_CORES, _RB, DH), lambda r: (0, r, 0)),
            pl.BlockSpec((NUM_CORES, _RB, DH), lambda r: (0, r, 0)),
        ],
        out_shape=[
            jax.ShapeDtypeStruct((NUM_CORES, N, DH), jnp.float32),
            jax.ShapeDtypeStruct((NUM_CORES, N, DH), jnp.float32),
        ],
    )(x, w0, w1)


NBUF = 4
NPBUF = 6
_UNROLL = 12


def _spmm_body(h0, h1, p0, p1, out, pbufs, bufs, psems, gsems, ssems, emb_sh):
    cid = lax.axis_index("c")
    sid = lax.axis_index("s")
    pbase = sid * NUM_CHUNKS

    bufs[0][...] = jnp.zeros_like(bufs[0])
    for t in range(-(-NZROWCH // NUM_SUBCORES)):
        j = sid + t * NUM_SUBCORES

        @pl.when(j < NZROWCH)
        def _():
            pltpu.sync_copy(bufs[0], emb_sh.at[pl.ds(j * ZCHUNK, ZCHUNK)])

    plsc.subcore_barrier()

    def do_support(h_ref, packed):
        def issue_pload(c, p):
            pltpu.async_copy(packed.at[pbase + c], pbufs[p], psems[p])

        def wait_pload(c, p):
            pltpu.make_async_copy(packed.at[pbase + c], pbufs[p],
                                  psems[p]).wait()

        def issue_gather(c, b, p):
            pltpu.async_copy(h_ref.at[pbufs[p].at[0]], bufs[b], gsems[b])

        def wait_gather(c, b, p):
            pltpu.make_async_copy(h_ref.at[pbufs[p].at[0]], bufs[b],
                                  gsems[b]).wait()

        def issue_scatter(c, b, p):
            pltpu.async_copy(bufs[b], emb_sh.at[pbufs[p].at[1]], ssems[b],
                             add=True)

        def wait_scatter(c, b, p):
            pltpu.make_async_copy(bufs[b], emb_sh.at[pbufs[p].at[1]],
                                  ssems[b]).wait()

        def scale(c, b, p):
            buf = bufs[b]
            vrow = pbufs[p]

            def scale_row(e, _):
                vi = vrow[2, pl.ds(e, 16)][0]
                val = jax.lax.bitcast_convert_type(vi, jnp.float32)
                for k in range(DH // 16):
                    sl = pl.ds(k * 16, 16)
                    buf[e, sl] = buf[e, sl] * val
                return ()

            lax.fori_loop(0, CHUNK, scale_row, ())

        def step(c, u, guarded, wait_sc, do_pload, do_gather):
            if wait_sc:
                if guarded:
                    @pl.when(c >= 2)
                    def _():
                        wait_scatter(c - 2, (u + 2) % NBUF, (u + 4) % NPBUF)
                else:
                    wait_scatter(c - 2, (u + 2) % NBUF, (u + 4) % NPBUF)
            if do_pload:
                issue_pload(c + 4, (u + 4) % NPBUF)
            if do_gather:
                wait_pload(c + 2, (u + 2) % NPBUF)
                issue_gather(c + 2, (u + 2) % NBUF, (u + 2) % NPBUF)
            wait_gather(c, u % NBUF, u % NPBUF)
            scale(c, u % NBUF, u % NPBUF)
            issue_scatter(c, u % NBUF, u % NPBUF)

        for j in range(4):
            issue_pload(j, j)
        for j in range(2):
            wait_pload(j, j)
            issue_gather(j, j, j)

        def group(i12, _):
            for u in range(_UNROLL):
                step(i12 * _UNROLL + u, u, True, True, True, True)
            return ()

        n_main = (NUM_CHUNKS - 4) // _UNROLL
        lax.fori_loop(0, n_main, group, ())
        for c in range(n_main * _UNROLL, NUM_CHUNKS):
            step(c, c % _UNROLL, False, True,
                 c + 4 < NUM_CHUNKS, c + 2 < NUM_CHUNKS)
        for c in range(NUM_CHUNKS - 2, NUM_CHUNKS):
            wait_scatter(c, c % NBUF, c % NPBUF)

    do_support(h0.at[cid], p0)
    do_support(h1.at[cid], p1)

    plsc.subcore_barrier()
    for t in range(-(-NZROWCH // NUM_SUBCORES)):
        j = sid + t * NUM_SUBCORES

        @pl.when(j < NZROWCH)
        def _():
            sl = pl.ds(j * ZCHUNK, ZCHUNK)
            pltpu.sync_copy(emb_sh.at[sl], out.at[cid].at[sl])


def _pack(rows, cols, vals):
    nch = NNZ // CHUNK
    return jnp.stack(
        [cols.reshape(nch, CHUNK),
         rows.reshape(nch, CHUNK),
         jax.lax.bitcast_convert_type(vals, jnp.int32).reshape(nch, CHUNK),
         jnp.zeros((nch, CHUNK), jnp.int32)],
        axis=1)


def _spmm(h0, h1, r0, c0, v0, r1, c1, v1):
    mesh = plsc.VectorSubcoreMesh(core_axis_name="c", subcore_axis_name="s")
    p0 = _pack(r0, c0, v0)
    p1 = _pack(r1, c1, v1)
    return pl.kernel(
        _spmm_body,
        out_type=jax.ShapeDtypeStruct((NUM_CORES, N, DH), jnp.float32),
        mesh=mesh,
        scratch_types=[
            [pltpu.VMEM((4, CHUNK), jnp.int32)] * NPBUF,
            [pltpu.VMEM((CHUNK, DH), jnp.float32)] * NBUF,
            [pltpu.SemaphoreType.DMA] * NPBUF,
            [pltpu.SemaphoreType.DMA] * NBUF,
            [pltpu.SemaphoreType.DMA] * NBUF,
            pltpu.VMEM_SHARED((N, DH), jnp.float32),
        ],
    )(h0, h1, p0, p1)


def _out_kernel(emb_ref, wd_ref, o_ref):
    e0 = jnp.maximum(emb_ref[0], 0.0)
    e1 = jnp.maximum(emb_ref[1], 0.0)
    wd0 = wd_ref[:, :DH]
    wd1 = wd_ref[:, DH:]
    dn = (((1,), (1,)), ((), ()))
    o_ref[...] = (jax.lax.dot_general(e0, wd0, dn)
                  + jax.lax.dot_general(e1, wd1, dn))


def _project(emb, wd):
    return pl.pallas_call(
        _out_kernel,
        grid=(N // _RB,),
        in_specs=[
            pl.BlockSpec((NUM_CORES, _RB, DH), lambda r: (0, r, 0)),
            pl.BlockSpec((D_DENSE, D_GCN), lambda r: (0, 0)),
        ],
        out_specs=pl.BlockSpec((_RB, D_DENSE), lambda r: (r, 0)),
        out_shape=jax.ShapeDtypeStruct((N, D_DENSE), jnp.float32),
    )(emb, wd)


@jax.jit
def kernel(user_X, item_X, W0, W1, Wd, A0_rows, A0_cols, A0_vals,
           A1_rows, A1_cols, A1_vals):
    x = jnp.concatenate([user_X, item_X], axis=0)
    h0, h1 = _features(x, W0, W1)
    emb = _spmm(h0, h1, A0_rows, A0_cols, A0_vals, A1_rows, A1_cols, A1_vals)
    out = _project(emb, Wd)
    return out[:N_USER], out[N_USER:]
